# Initial kernel scaffold; baseline (speedup 1.0000x reference)
import jax
import jax.numpy as jnp
from jax.experimental import pallas as pl


def kernel(x):
    raise NotImplementedError("write your pallas kernel here")


# speedup vs baseline: 109.8299x; 109.8299x over previous
import jax
import jax.numpy as jnp
from jax.experimental import pallas as pl
from jax.experimental.pallas import tpu as pltpu

_BS = 256


def _min_kernel(x_ref, min_ref):
    i = pl.program_id(0)
    sal = jnp.abs(jnp.mean(x_ref[...], axis=0))
    bmin = jnp.min(sal)

    @pl.when(i == 0)
    def _init():
        min_ref[0, 0] = bmin

    @pl.when(i > 0)
    def _acc():
        min_ref[0, 0] = jnp.minimum(min_ref[0, 0], bmin)


def _apply_kernel(thr_ref, x_ref, o_ref):
    thr = thr_ref[0]
    x = x_ref[...]
    sal = jnp.abs(jnp.mean(x, axis=0))
    mask = (sal >= thr).astype(x.dtype)
    o_ref[...] = x * mask[None, :, :]


def kernel(x):
    b, s, d = x.shape
    grid = (s // _BS,)

    thr = pl.pallas_call(
        _min_kernel,
        grid=grid,
        in_specs=[pl.BlockSpec((b, _BS, d), lambda i: (0, i, 0))],
        out_specs=pl.BlockSpec(memory_space=pltpu.SMEM),
        out_shape=jax.ShapeDtypeStruct((1, 1), x.dtype),
        compiler_params=pltpu.CompilerParams(
            dimension_semantics=("arbitrary",),
        ),
    )(x)

    out = pl.pallas_call(
        _apply_kernel,
        grid=grid,
        in_specs=[
            pl.BlockSpec(memory_space=pltpu.SMEM),
            pl.BlockSpec((b, _BS, d), lambda i: (0, i, 0)),
        ],
        out_specs=pl.BlockSpec((b, _BS, d), lambda i: (0, i, 0)),
        out_shape=jax.ShapeDtypeStruct((b, s, d), x.dtype),
        compiler_params=pltpu.CompilerParams(
            dimension_semantics=("parallel",),
        ),
    )(thr.reshape(-1), x)
    return out


# speedup vs baseline: 162.3400x; 1.4781x over previous
import jax
import jax.numpy as jnp
from jax.experimental import pallas as pl
from jax.experimental.pallas import tpu as pltpu

_BS = 256


def _prune_kernel(x_ref, o_ref):
    x = x_ref[...]
    sal = jnp.abs(jnp.mean(x, axis=0))
    thr = jnp.min(sal)
    mask = (sal >= thr).astype(x.dtype)
    o_ref[...] = x * mask[None, :, :]


def kernel(x):
    b, s, d = x.shape
    grid = (s // _BS,)
    return pl.pallas_call(
        _prune_kernel,
        grid=grid,
        in_specs=[pl.BlockSpec((b, _BS, d), lambda i: (0, i, 0))],
        out_specs=pl.BlockSpec((b, _BS, d), lambda i: (0, i, 0)),
        out_shape=jax.ShapeDtypeStruct((b, s, d), x.dtype),
        compiler_params=pltpu.CompilerParams(
            dimension_semantics=("parallel",),
        ),
    )(x)
